# Initial kernel scaffold; baseline (speedup 1.0000x reference)
import jax
import jax.numpy as jnp
from jax.experimental import pallas as pl


def kernel(x, Wg, W1, b1, W2, b2):
    raise NotImplementedError("write your pallas kernel here")


# speedup vs baseline: 50.5195x; 50.5195x over previous
import functools
import math

import jax
import jax.numpy as jnp
from jax import lax
from jax.experimental import pallas as pl
from jax.experimental.pallas import tpu as pltpu
from jax.experimental.pallas import tpu_sc as plsc

NUM_EXPERTS = 8
TOP_K = 2
CAPACITY_FACTOR = 1.25

NC = 2
NS = 16
NW = NC * NS
LANES = 16


def _router_body(T, E, CAP, CAP_PAD, x_ref, wg_ref, e_ref, p_ref, w_ref):
    xv = x_ref[...]
    wg = wg_ref[...]
    logits = jnp.dot(xv, wg, preferred_element_type=jnp.float32)
    m = jnp.max(logits, axis=1, keepdims=True)
    ex = jnp.exp(logits - m)
    probs = ex / jnp.sum(ex, axis=1, keepdims=True)
    lane = lax.broadcasted_iota(jnp.int32, (T, E), 1)
    w0 = jnp.max(probs, axis=1, keepdims=True)
    e0 = jnp.min(jnp.where(probs == w0, lane, E), axis=1, keepdims=True)
    probs1 = jnp.where(lane == e0, -1.0, probs)
    w1 = jnp.max(probs1, axis=1, keepdims=True)
    e1 = jnp.min(jnp.where(probs1 == w1, lane, E), axis=1, keepdims=True)
    denom = w0 + w1 + 1e-8
    w0n = w0 / denom
    w1n = w1 / denom
    oh0 = (lane == e0).astype(jnp.float32)
    oh1 = (lane == e1).astype(jnp.float32)
    s = oh0 + oh1
    c = s
    k = 1
    while k < T:
        c = c + jnp.concatenate([jnp.zeros((k, E), jnp.float32), c[: T - k]], axis=0)
        k *= 2
    excl = c - s
    pos0 = jnp.sum(oh0 * excl, axis=1, keepdims=True).astype(jnp.int32)
    pos1 = jnp.sum(oh1 * excl, axis=1, keepdims=True).astype(jnp.int32)
    valid0 = pos0 < CAP
    valid1 = pos1 < CAP
    p0 = jnp.where(valid0, pos0, CAP)
    p1 = jnp.where(valid1, pos1, CAP)
    we0 = jnp.where(valid0, w0n, 0.0)
    we1 = jnp.where(valid1, w1n, 0.0)
    e_ref[...] = jnp.concatenate([e0, e1], axis=1)
    p_ref[...] = jnp.concatenate([p0, p1], axis=1)
    w_ref[...] = jnp.concatenate([we0, we1], axis=1)


def _router(xf, Wg, CAP, CAP_PAD):
    T, D = xf.shape
    E = Wg.shape[1]
    return pl.pallas_call(
        functools.partial(_router_body, T, E, CAP, CAP_PAD),
        out_shape=(
            jax.ShapeDtypeStruct((T, TOP_K), jnp.int32),
            jax.ShapeDtypeStruct((T, TOP_K), jnp.int32),
            jax.ShapeDtypeStruct((T, TOP_K), jnp.float32),
        ),
    )(xf, Wg)


def _dispatch(ef, pf, wf, xf, CAP_PAD):
    NI = ef.shape[0]
    T, D = xf.shape
    E = NUM_EXPERTS
    TOTAL = E * CAP_PAD
    ROWS = TOTAL // NW
    CHUNK = ROWS // 3 if ROWS % 3 == 0 else ROWS
    mesh = plsc.VectorSubcoreMesh(core_axis_name="c", subcore_axis_name="s")

    @functools.partial(
        pl.kernel,
        out_type=(
            jax.ShapeDtypeStruct((TOTAL,), jnp.float32),
            jax.ShapeDtypeStruct((TOTAL, D), jnp.float32),
        ),
        mesh=mesh,
        scratch_types=[
            pltpu.VMEM((NI,), jnp.int32),
            pltpu.VMEM((NI,), jnp.int32),
            pltpu.VMEM((NI,), jnp.float32),
            pltpu.VMEM((TOTAL,), jnp.int32),
            pltpu.VMEM((TOTAL,), jnp.float32),
            pltpu.VMEM((CHUNK, D), jnp.float32),
            pltpu.SemaphoreType.DMA,
        ],
        compiler_params=pltpu.CompilerParams(needs_layout_passes=False),
    )
    def body(e_hbm, p_hbm, w_hbm, x_hbm, wt_hbm, xin_hbm,
             ev, pv, wv, src_v, wt_v, rows_v, sem):
        cid = lax.axis_index("c")
        sid = lax.axis_index("s")
        wid = sid * NC + cid
        pltpu.sync_copy(e_hbm, ev)
        pltpu.sync_copy(p_hbm, pv)
        pltpu.sync_copy(w_hbm, wv)

        zi = jnp.zeros((LANES,), jnp.int32)
        zf = jnp.zeros((LANES,), jnp.float32)

        def zloop(i, carry):
            src_v[pl.ds(i * LANES, LANES)] = zi
            wt_v[pl.ds(i * LANES, LANES)] = zf
            return carry

        lax.fori_loop(0, TOTAL // LANES, zloop, 0)

        def sloop(i, carry):
            base = i * LANES
            e16 = ev[pl.ds(base, LANES)]
            p16 = pv[pl.ds(base, LANES)]
            w16 = wv[pl.ds(base, LANES)]
            tok16 = lax.shift_right_logical(base + lax.iota(jnp.int32, LANES), 1)
            slot16 = e16 * CAP_PAD + p16
            idxt16 = p16 * E + e16
            plsc.store_scatter(src_v, [slot16], tok16)
            plsc.store_scatter(wt_v, [idxt16], w16)
            return carry

        lax.fori_loop(0, NI // LANES, sloop, 0)

        @pl.when(jnp.logical_and(cid == 0, sid == 0))
        def _():
            pltpu.sync_copy(wt_v, wt_hbm)

        for ci in range(ROWS // CHUNK):
            base = wid * ROWS + ci * CHUNK
            idx_ref = src_v.at[pl.ds(base, CHUNK)]
            pltpu.async_copy(x_hbm.at[idx_ref], rows_v, sem).wait()
            pltpu.sync_copy(rows_v, xin_hbm.at[pl.ds(base, CHUNK)])

    return body(ef, pf, wf, xf)


def _ffn_body(NH, xin_ref, w1_ref, b1_ref, w2_ref, b2_ref, wt_ref, out_ref):
    e_idx = pl.program_id(0)
    h_idx = pl.program_id(1)
    xv = xin_ref[0]
    h = jnp.dot(xv, w1_ref[0], preferred_element_type=jnp.float32) + b1_ref[0]
    h = jnp.maximum(h, 0.0)
    part = jnp.dot(h, w2_ref[0], preferred_element_type=jnp.float32)

    @pl.when(h_idx == 0)
    def _():
        out_ref[0] = part + b2_ref[0]

    @pl.when(h_idx > 0)
    def _():
        out_ref[0] = out_ref[0] + part

    @pl.when(h_idx == NH - 1)
    def _():
        E = wt_ref.shape[1]
        oh = (lax.broadcasted_iota(jnp.int32, (E, 1), 0) == e_idx).astype(jnp.float32)
        wcol = jnp.dot(wt_ref[...], oh, preferred_element_type=jnp.float32)
        out_ref[0] = out_ref[0] * wcol


def _ffn(xin, W1, b1, W2, b2, wt):
    E, CAP_PAD, D = xin.shape
    H = W1.shape[2]
    NH = 4
    HB = H // NH
    grid = (E, NH)
    return pl.pallas_call(
        functools.partial(_ffn_body, NH),
        grid=grid,
        in_specs=[
            pl.BlockSpec((1, CAP_PAD, D), lambda e, h: (e, 0, 0)),
            pl.BlockSpec((1, D, HB), lambda e, h: (e, 0, h)),
            pl.BlockSpec((1, 1, HB), lambda e, h: (e, 0, h)),
            pl.BlockSpec((1, HB, D), lambda e, h: (e, h, 0)),
            pl.BlockSpec((1, 1, D), lambda e, h: (e, 0, 0)),
            pl.BlockSpec((CAP_PAD, E), lambda e, h: (0, 0)),
        ],
        out_specs=pl.BlockSpec((1, CAP_PAD, D), lambda e, h: (e, 0, 0)),
        out_shape=jax.ShapeDtypeStruct((E, CAP_PAD, D), jnp.float32),
        compiler_params=pltpu.CompilerParams(
            dimension_semantics=("arbitrary", "arbitrary"),
        ),
    )(xin, W1, b1.reshape(E, 1, H), W2, b2.reshape(E, 1, D), wt)


def _combine(ef, pf, yw, T, CAP_PAD):
    NI = ef.shape[0]
    D = yw.shape[1]
    TPW = T // NW
    TCHUNK = 32
    mesh = plsc.VectorSubcoreMesh(core_axis_name="c", subcore_axis_name="s")

    @functools.partial(
        pl.kernel,
        out_type=jax.ShapeDtypeStruct((T, D), jnp.float32),
        mesh=mesh,
        scratch_types=[
            pltpu.VMEM((NI,), jnp.int32),
            pltpu.VMEM((NI,), jnp.int32),
            pltpu.VMEM((TCHUNK,), jnp.int32),
            pltpu.VMEM((TCHUNK,), jnp.int32),
            pltpu.VMEM((TCHUNK, D), jnp.float32),
            pltpu.VMEM((TCHUNK, D), jnp.float32),
            pltpu.SemaphoreType.DMA,
            pltpu.SemaphoreType.DMA,
        ],
        compiler_params=pltpu.CompilerParams(needs_layout_passes=False),
    )
    def body(e_hbm, p_hbm, yw_hbm, out_hbm,
             ev, pv, idx0_v, idx1_v, buf0, buf1, sem0, sem1):
        cid = lax.axis_index("c")
        sid = lax.axis_index("s")
        wid = sid * NC + cid
        pltpu.sync_copy(e_hbm, ev)
        pltpu.sync_copy(p_hbm, pv)
        for ci in range(TPW // TCHUNK):
            tokbase = wid * TPW + ci * TCHUNK
            for c2 in range(TCHUNK // LANES):
                tok16 = tokbase + c2 * LANES + lax.iota(jnp.int32, LANES)
                it0 = tok16 * TOP_K
                it1 = it0 + 1
                s0 = (plsc.load_gather(ev, [it0]) * CAP_PAD
                      + plsc.load_gather(pv, [it0]))
                s1 = (plsc.load_gather(ev, [it1]) * CAP_PAD
                      + plsc.load_gather(pv, [it1]))
                idx0_v[pl.ds(c2 * LANES, LANES)] = s0
                idx1_v[pl.ds(c2 * LANES, LANES)] = s1
            cp0 = pltpu.async_copy(yw_hbm.at[idx0_v], buf0, sem0)
            cp1 = pltpu.async_copy(yw_hbm.at[idx1_v], buf1, sem1)
            cp0.wait()
            cp1.wait()
            for r in range(TCHUNK):
                def aloop(c, carry, r=r):
                    col = c * (4 * LANES)
                    for u in range(4):
                        off = col + u * LANES
                        buf0[r, pl.ds(off, LANES)] = (
                            buf0[r, pl.ds(off, LANES)] + buf1[r, pl.ds(off, LANES)])
                    return carry
                lax.fori_loop(0, D // (4 * LANES), aloop, 0)
            pltpu.sync_copy(buf0, out_hbm.at[pl.ds(tokbase, TCHUNK)])

    return body(ef, pf, yw)


def kernel(x, Wg, W1, b1, W2, b2):
    B, S, D = x.shape
    T = B * S
    E = Wg.shape[1]
    NI = T * TOP_K
    CAP = int(math.ceil(NI / E * CAPACITY_FACTOR))
    CAP_PAD = CAP + 32

    xf = x.reshape(T, D)
    e2, p2, w2 = _router(xf, Wg, CAP, CAP_PAD)
    ef = e2.reshape(NI)
    pf = p2.reshape(NI)
    wf = w2.reshape(NI)
    wt, xin = _dispatch(ef, pf, wf, xf, CAP_PAD)
    yw = _ffn(xin.reshape(E, CAP_PAD, D), W1, b1, W2, b2,
              wt.reshape(CAP_PAD, E))
    out = _combine(ef, pf, yw.reshape(E * CAP_PAD, D), T, CAP_PAD)
    return out.reshape(B, S, D)
